# Initial kernel scaffold; baseline (speedup 1.0000x reference)
import jax
import jax.numpy as jnp
from jax.experimental import pallas as pl


def kernel(inputs, embedding):
    raise NotImplementedError("write your pallas kernel here")


# speedup vs baseline: 10.1090x; 10.1090x over previous
import functools

import jax
import jax.numpy as jnp
from jax import lax
from jax.experimental import pallas as pl
from jax.experimental.pallas import tpu as pltpu
from jax.experimental.pallas import tpu_sc as plsc

B, C, H, W = 16, 256, 32, 32
HW = H * W
N_TOK = B * HW
N_EMB = 8192
E_TILE = 2048
N_ETILES = N_EMB // E_TILE
COMMITMENT_COST = 0.25

NC = 2
NS = 16
NW = NC * NS
TOK_PER_W = N_TOK // NW
GCHUNK = 128
NCH = TOK_PER_W // GCHUNK


def _argmin_body(x_ref, emb_ref, idx_ref, x2_ref, mval_ref, midx_ref):
    e = pl.program_id(1)
    xb = x_ref[0]

    @pl.when(e == 0)
    def _():
        x2_ref[...] = jnp.sum(xb * xb, axis=0, keepdims=True)

    emb = emb_ref[...]
    mm = lax.dot_general(emb.astype(jnp.bfloat16), xb.astype(jnp.bfloat16),
                         (((1,), (0,)), ((), ())),
                         preferred_element_type=jnp.float32)
    e2 = jnp.sum(emb * emb, axis=1, keepdims=True)
    d = (x2_ref[...] + e2) - 2.0 * mm

    tmin = jnp.min(d, axis=0, keepdims=True)
    rows = lax.broadcasted_iota(jnp.int32, (E_TILE, HW), 0)
    targ = jnp.min(jnp.where(d == tmin, rows, N_EMB), axis=0, keepdims=True)
    targ = targ + e * E_TILE

    @pl.when(e == 0)
    def _():
        mval_ref[...] = tmin.astype(jnp.bfloat16).astype(jnp.float32)
        midx_ref[...] = targ

    @pl.when(e > 0)
    def _():
        better = tmin < mval_ref[...]
        mval_ref[...] = jnp.where(
            better, tmin, mval_ref[...]).astype(jnp.bfloat16).astype(jnp.float32)
        midx_ref[...] = jnp.where(better, targ, midx_ref[...])

    @pl.when(e == N_ETILES - 1)
    def _():
        idx_ref[0] = midx_ref[...]


def _argmin_call(x3, emb):
    return pl.pallas_call(
        _argmin_body,
        grid=(B, N_ETILES),
        in_specs=[
            pl.BlockSpec((1, C, HW), lambda b, e: (b, 0, 0)),
            pl.BlockSpec((E_TILE, C), lambda b, e: (e, 0)),
        ],
        out_specs=pl.BlockSpec((1, 1, HW), lambda b, e: (b, 0, 0)),
        out_shape=jax.ShapeDtypeStruct((B, 1, HW), jnp.int32),
        scratch_shapes=[
            pltpu.VMEM((1, HW), jnp.float32),
            pltpu.VMEM((1, HW), jnp.float32),
            pltpu.VMEM((1, HW), jnp.int32),
        ],
        compiler_params=pltpu.CompilerParams(
            dimension_semantics=("arbitrary", "arbitrary")),
    )(x3, emb)


def _sc_body(emb_hbm, idx_hbm, zeros_hbm,
             q_hbm, counts_hbm,
             idx_v, rows_v, counts_v, sem):
    c = lax.axis_index("c")
    s = lax.axis_index("s")
    wid = s * NC + c
    base = wid * TOK_PER_W

    pltpu.sync_copy(idx_hbm.at[pl.ds(base, TOK_PER_W)], idx_v)
    pltpu.sync_copy(zeros_hbm, counts_v)

    for j in range(NCH):
        idx_slice = idx_v.at[pl.ds(j * GCHUNK, GCHUNK)]
        pltpu.async_copy(emb_hbm.at[idx_slice], rows_v, sem).wait()
        pltpu.sync_copy(rows_v, q_hbm.at[pl.ds(base + j * GCHUNK, GCHUNK)])

    ones16 = jnp.full((16,), 1, jnp.int32)

    def hist_step(i, carry):
        v = idx_v[pl.ds(i * 16, 16)]
        plsc.addupdate_scatter(counts_v, [v], ones16)
        return carry

    lax.fori_loop(0, TOK_PER_W // 16, hist_step, 0)
    pltpu.sync_copy(counts_v, counts_hbm.at[wid])


def _sc_call(emb, idx_flat, zeros_z):
    mesh = plsc.VectorSubcoreMesh(core_axis_name="c", subcore_axis_name="s")
    f = pl.kernel(
        _sc_body,
        out_type=[
            jax.ShapeDtypeStruct((N_TOK, C), jnp.float32),
            jax.ShapeDtypeStruct((NW, N_EMB), jnp.int32),
        ],
        mesh=mesh,
        scratch_types=[
            pltpu.VMEM((TOK_PER_W,), jnp.int32),
            pltpu.VMEM((GCHUNK, C), jnp.float32),
            pltpu.VMEM((N_EMB,), jnp.int32),
            pltpu.SemaphoreType.DMA,
        ],
        compiler_params=pltpu.CompilerParams(needs_layout_passes=False),
    )
    return f(emb, idx_flat, zeros_z)


def _finalize_body(q_ref, x_ref, counts_ref, loss_ref, out_ref, perp_ref,
                   acc_ref):
    b = pl.program_id(0)
    qb = q_ref[0]
    xb = x_ref[0]
    qt = qb.T
    diff = qt - xb
    out_ref[0] = xb + diff

    @pl.when(b == 0)
    def _():
        acc_ref[0, 0] = 0.0

    acc_ref[0, 0] += jnp.sum(diff * diff)

    @pl.when(b == B - 1)
    def _():
        loss = (COMMITMENT_COST / (N_TOK * C)) * acc_ref[0, 0]
        loss_ref[...] = jnp.reshape(loss, (1, 1))
        cnt = counts_ref[...].astype(jnp.float32)
        tot = jnp.sum(cnt, axis=0, keepdims=True)
        p = tot * (1.0 / N_TOK)
        ent = jnp.sum(p * jnp.log(p + 1e-10))
        perp_ref[...] = jnp.reshape(jnp.exp(-ent), (1, 1))


def _finalize_call(q3, x3, counts):
    return pl.pallas_call(
        _finalize_body,
        grid=(B,),
        in_specs=[
            pl.BlockSpec((1, HW, C), lambda b: (b, 0, 0)),
            pl.BlockSpec((1, C, HW), lambda b: (b, 0, 0)),
            pl.BlockSpec((NW, N_EMB), lambda b: (0, 0)),
        ],
        out_specs=[
            pl.BlockSpec((1, 1), lambda b: (0, 0)),
            pl.BlockSpec((1, C, HW), lambda b: (b, 0, 0)),
            pl.BlockSpec((1, 1), lambda b: (0, 0)),
        ],
        out_shape=[
            jax.ShapeDtypeStruct((1, 1), jnp.float32),
            jax.ShapeDtypeStruct((B, C, HW), jnp.float32),
            jax.ShapeDtypeStruct((1, 1), jnp.float32),
        ],
        scratch_shapes=[pltpu.SMEM((1, 1), jnp.float32)],
        compiler_params=pltpu.CompilerParams(
            dimension_semantics=("arbitrary",)),
    )(q3, x3, counts)


def kernel(inputs, embedding):
    x3 = inputs.reshape(B, C, HW)
    idx2d = _argmin_call(x3, embedding)

    zeros_z = jnp.zeros((N_EMB,), jnp.int32)
    q, counts = _sc_call(embedding, idx2d.reshape(N_TOK), zeros_z)

    loss11, qout, perp11 = _finalize_call(q.reshape(B, HW, C), x3, counts)

    return (loss11.reshape(()), qout.reshape(B, C, H, W),
            perp11.reshape(()), idx2d.reshape(B, H, W))
